# baseline (speedup 1.0000x reference)
import functools
import math

import jax
import jax.numpy as jnp
from jax import lax
from jax.experimental import pallas as pl
from jax.experimental.pallas import tpu as pltpu
from jax.experimental.pallas import tpu_sc as plsc

N, E, D, R = 10000, 320000, 128, 16

NUM_CORES = 2
NUM_SUBCORES = 16
NW = NUM_CORES * NUM_SUBCORES
E_PER_W = E // NW
B = 80
N_CHUNKS = E_PER_W // B
E_ROWS = E // D


N_TILES = 5
N_TILE = N // N_TILES


def _tc_body(emb_ref, wk_ref, wq_ref, rel_ref, si_ref, p_ref,
             b_ref, ai_ref, k_ref):
    r = pl.program_id(0)
    nt = pl.program_id(1)
    rows = pl.ds(nt * N_TILE, N_TILE)

    @pl.when(r == 0)
    def _():
        scale = 1.0 / math.sqrt(D)
        dn = (((1,), (1,)), ((), ()))
        k = lax.dot_general(emb_ref[...], wk_ref[...], dn,
                            precision=lax.Precision.HIGHEST,
                            preferred_element_type=jnp.float32)
        k_ref[rows, :] = k * scale

    @pl.when((r == 0) & (nt == 0))
    def _():
        ai_ref[...] = p_ref[...] * N + si_ref[...]

    m = k_ref[rows, :] * rel_ref[r, :][None, :]
    dn = (((1,), (0,)), ((), ()))
    b_ref[...] = lax.dot_general(m, wq_ref[...], dn,
                                 precision=lax.Precision.HIGHEST,
                                 preferred_element_type=jnp.float32)


def _tc_stage(node_emb, tokeys, toqueries, relations, si2, p2):
    return pl.pallas_call(
        _tc_body,
        grid=(R, N_TILES),
        in_specs=[pl.BlockSpec((N_TILE, D), lambda r, nt: (nt, 0)),
                  pl.BlockSpec((D, D), lambda r, nt: (0, 0)),
                  pl.BlockSpec((D, D), lambda r, nt: (0, 0)),
                  pl.BlockSpec((R, D), lambda r, nt: (0, 0)),
                  pl.BlockSpec((E_ROWS, D), lambda r, nt: (0, 0)),
                  pl.BlockSpec((E_ROWS, D), lambda r, nt: (0, 0))],
        out_specs=[pl.BlockSpec((N_TILE, D), lambda r, nt: (r * N_TILES + nt, 0)),
                   pl.BlockSpec((E_ROWS, D), lambda r, nt: (0, 0))],
        out_shape=[jax.ShapeDtypeStruct((R * N, D), jnp.float32),
                   jax.ShapeDtypeStruct((E_ROWS, D), jnp.int32)],
        scratch_shapes=[pltpu.VMEM((N, D), jnp.float32)],
    )(node_emb, tokeys, toqueries, relations, si2, p2)


def _sc_body(b_hbm, emb_hbm, ai_hbm, oi_hbm,
             dots_hbm, newemb_hbm,
             ai_v, oi_v,
             br0, br1, er0, er1,
             tmp_v, dots_v, gs0, gs1, ws0, ws1):
    wid = lax.axis_index("s") * NUM_CORES + lax.axis_index("c")
    wbase = wid * E_PER_W
    lanes = lax.iota(jnp.int32, 16)

    pltpu.sync_copy(ai_hbm.at[pl.ds(wbase, E_PER_W)], ai_v)
    pltpu.sync_copy(oi_hbm.at[pl.ds(wbase, E_PER_W)], oi_v)

    slots = ((br0, er0, gs0, ws0), (br1, er1, gs1, ws1))

    def fire(g, slot):
        br, er, gs, _ = slot
        off = g * B
        pltpu.async_copy(b_hbm.at[ai_v.at[pl.ds(off, B)]], br, gs)
        pltpu.async_copy(emb_hbm.at[oi_v.at[pl.ds(off, B)]], er, gs)

    def drain_gathers(slot):
        br, er, gs, _ = slot
        pltpu.make_async_copy(b_hbm.at[pl.ds(0, B)], br, gs).wait()
        pltpu.make_async_copy(emb_hbm.at[pl.ds(0, B)], er, gs).wait()

    def fire_wb(g, slot):
        _, er, _, ws = slot
        pltpu.async_copy(er, newemb_hbm.at[pl.ds(wbase + g * B, B)], ws)

    def wait_wb(g, slot):
        _, er, _, ws = slot
        pltpu.make_async_copy(er, newemb_hbm.at[pl.ds(wbase + g * B, B)],
                              ws).wait()

    def compute(g, slot):
        br, er = slot[0], slot[1]
        goff = g * B

        @plsc.parallel_loop(0, B, step=16)
        def e_body(t):
            for l in range(16):
                e = t + l
                acc = br[e, pl.ds(0, 16)] * er[e, pl.ds(0, 16)]
                for c in range(1, 8):
                    acc = acc + (br[e, pl.ds(c * 16, 16)] *
                                 er[e, pl.ds(c * 16, 16)])
                tmp_v[pl.ds(e * 17, 16)] = acc
            base17 = (t + lanes) * 17
            tot = plsc.load_gather(tmp_v, [base17])
            for c in range(1, 16):
                tot = tot + plsc.load_gather(tmp_v, [base17 + c])
            dots_v[pl.ds(goff + t, 16)] = tot

    fire(0, slots[0])
    fire(1, slots[1])

    def pair_body(v, _):
        g = v * 2
        for par in range(2):
            slot = slots[par]
            drain_gathers(slot)
            fire_wb(g + par, slot)
            compute(g + par, slot)
            wait_wb(g + par, slot)
            fire(g + par + 2, slot)
        return 0

    lax.fori_loop(0, (N_CHUNKS - 3) // 2, pair_body, 0)
    g_tail = N_CHUNKS - 3
    for g in range(g_tail, N_CHUNKS):
        slot = slots[g % 2]
        drain_gathers(slot)
        fire_wb(g, slot)
        compute(g, slot)
        wait_wb(g, slot)
        if g + 2 < N_CHUNKS:
            fire(g + 2, slot)

    pltpu.sync_copy(dots_v, dots_hbm.at[pl.ds(wbase, E_PER_W)])


def _sc_call(b_tab, node_emb, ai, oi):
    mesh = plsc.VectorSubcoreMesh(core_axis_name="c", subcore_axis_name="s",
                                  num_cores=NUM_CORES,
                                  num_subcores=NUM_SUBCORES)
    f = pl.kernel(
        _sc_body,
        out_type=[jax.ShapeDtypeStruct((E,), jnp.float32),
                  jax.ShapeDtypeStruct((E, D), jnp.float32)],
        mesh=mesh,
        compiler_params=pltpu.CompilerParams(needs_layout_passes=False),
        scratch_types=[
            pltpu.VMEM((E_PER_W,), jnp.int32),
            pltpu.VMEM((E_PER_W,), jnp.int32),
            pltpu.VMEM((B, D), jnp.float32),
            pltpu.VMEM((B, D), jnp.float32),
            pltpu.VMEM((B, D), jnp.float32),
            pltpu.VMEM((B, D), jnp.float32),
            pltpu.VMEM((B * 17,), jnp.float32),
            pltpu.VMEM((E_PER_W,), jnp.float32),
            pltpu.SemaphoreType.DMA,
            pltpu.SemaphoreType.DMA,
            pltpu.SemaphoreType.DMA,
            pltpu.SemaphoreType.DMA,
        ],
    )
    return f(b_tab, node_emb, ai, oi)


def kernel(node_emb, edge_index, edge_type, relations, tokeys, toqueries):
    si2 = edge_index[0].reshape(E_ROWS, D)
    p2 = edge_type.reshape(E_ROWS, D)
    b_tab, ai2 = _tc_stage(node_emb, tokeys, toqueries, relations, si2, p2)
    ai = ai2.reshape(E)
    oi = edge_index[1]
    dots, new_node_emb = _sc_call(b_tab, node_emb, ai, oi)
    return dots, new_node_emb


# speedup vs baseline: 1.2092x; 1.2092x over previous
import functools
import math

import jax
import jax.numpy as jnp
from jax import lax
from jax.experimental import pallas as pl
from jax.experimental.pallas import tpu as pltpu
from jax.experimental.pallas import tpu_sc as plsc

N, E, D, R = 10000, 320000, 128, 16

NUM_CORES = 2
NUM_SUBCORES = 16
NW = NUM_CORES * NUM_SUBCORES
E_PER_W = E // NW
B = 80
N_CHUNKS = E_PER_W // B
E_ROWS = E // D


def _proj_body(emb_ref, wk_ref, si_ref, p_ref, k_ref, ai_ref):
    scale = 1.0 / math.sqrt(D)
    dn = (((1,), (1,)), ((), ()))
    k = lax.dot_general(emb_ref[...], wk_ref[...], dn,
                        precision=lax.Precision.HIGHEST,
                        preferred_element_type=jnp.float32)
    k_ref[...] = k * scale
    ai_ref[...] = p_ref[...] * N + si_ref[...]


def _project(node_emb, tokeys, si2, p2):
    return pl.pallas_call(
        _proj_body,
        out_shape=[jax.ShapeDtypeStruct((N, D), jnp.float32),
                   jax.ShapeDtypeStruct((E_ROWS, D), jnp.int32)],
    )(node_emb, tokeys, si2, p2)


def _expand_body(k_ref, rel_ref, wq_ref, b_ref):
    r = pl.program_id(0)
    m = k_ref[...] * rel_ref[r, :][None, :]
    dn = (((1,), (0,)), ((), ()))
    b_ref[...] = lax.dot_general(m, wq_ref[...], dn,
                                 precision=lax.Precision.HIGHEST,
                                 preferred_element_type=jnp.float32)


def _expand(k_tab, relations, toqueries):
    return pl.pallas_call(
        _expand_body,
        grid=(R,),
        in_specs=[pl.BlockSpec((N, D), lambda r: (0, 0)),
                  pl.BlockSpec((R, D), lambda r: (0, 0)),
                  pl.BlockSpec((D, D), lambda r: (0, 0))],
        out_specs=pl.BlockSpec((N, D), lambda r: (r, 0)),
        out_shape=jax.ShapeDtypeStruct((R * N, D), jnp.float32),
    )(k_tab, relations, toqueries)


def _sc_body(b_hbm, emb_hbm, ai_hbm, oi_hbm,
             dots_hbm, newemb_hbm,
             ai_v, oi_v,
             br0, br1, br2, er0, er1, er2,
             tmp_v, dots_v, gs0, gs1, gs2, ws0, ws1, ws2):
    wid = lax.axis_index("s") * NUM_CORES + lax.axis_index("c")
    wbase = wid * E_PER_W
    lanes = lax.iota(jnp.int32, 16)

    pltpu.sync_copy(ai_hbm.at[pl.ds(wbase, E_PER_W)], ai_v)
    pltpu.sync_copy(oi_hbm.at[pl.ds(wbase, E_PER_W)], oi_v)

    slots = ((br0, er0, gs0, ws0), (br1, er1, gs1, ws1),
             (br2, er2, gs2, ws2))

    def fire(g, slot):
        br, er, gs, _ = slot
        off = g * B
        pltpu.async_copy(b_hbm.at[ai_v.at[pl.ds(off, B)]], br, gs)
        pltpu.async_copy(emb_hbm.at[oi_v.at[pl.ds(off, B)]], er, gs)

    def drain_gathers(slot):
        br, er, gs, _ = slot
        pltpu.make_async_copy(b_hbm.at[pl.ds(0, B)], br, gs).wait()
        pltpu.make_async_copy(emb_hbm.at[pl.ds(0, B)], er, gs).wait()

    def fire_wb(g, slot):
        _, er, _, ws = slot
        pltpu.async_copy(er, newemb_hbm.at[pl.ds(wbase + g * B, B)], ws)

    def wait_wb(g, slot):
        _, er, _, ws = slot
        pltpu.make_async_copy(er, newemb_hbm.at[pl.ds(wbase + g * B, B)],
                              ws).wait()

    def compute(g, slot):
        br, er = slot[0], slot[1]
        goff = g * B

        @plsc.parallel_loop(0, B, step=16)
        def e_body(t):
            for l in range(16):
                e = t + l
                acc = br[e, pl.ds(0, 16)] * er[e, pl.ds(0, 16)]
                for c in range(1, 8):
                    acc = acc + (br[e, pl.ds(c * 16, 16)] *
                                 er[e, pl.ds(c * 16, 16)])
                tmp_v[pl.ds(e * 17, 16)] = acc
            base17 = (t + lanes) * 17
            tot = plsc.load_gather(tmp_v, [base17])
            for c in range(1, 16):
                tot = tot + plsc.load_gather(tmp_v, [base17 + c])
            dots_v[pl.ds(goff + t, 16)] = tot

    fire(0, slots[0])
    fire(1, slots[1])
    fire(2, slots[2])

    def triple_body(v, _):
        g = v * 3
        for par in range(3):
            slot = slots[par]
            drain_gathers(slot)
            fire_wb(g + par, slot)
            compute(g + par, slot)
            wait_wb(g + par, slot)
            fire(g + par + 3, slot)
        return 0

    n_steady = (N_CHUNKS - 5) // 3
    lax.fori_loop(0, n_steady, triple_body, 0)
    for g in range(n_steady * 3, N_CHUNKS):
        slot = slots[g % 3]
        drain_gathers(slot)
        fire_wb(g, slot)
        compute(g, slot)
        wait_wb(g, slot)
        if g + 3 < N_CHUNKS:
            fire(g + 3, slot)

    pltpu.sync_copy(dots_v, dots_hbm.at[pl.ds(wbase, E_PER_W)])


def _sc_call(b_tab, node_emb, ai, oi):
    mesh = plsc.VectorSubcoreMesh(core_axis_name="c", subcore_axis_name="s",
                                  num_cores=NUM_CORES,
                                  num_subcores=NUM_SUBCORES)
    f = pl.kernel(
        _sc_body,
        out_type=[jax.ShapeDtypeStruct((E,), jnp.float32),
                  jax.ShapeDtypeStruct((E, D), jnp.float32)],
        mesh=mesh,
        compiler_params=pltpu.CompilerParams(needs_layout_passes=False),
        scratch_types=[
            pltpu.VMEM((E_PER_W,), jnp.int32),
            pltpu.VMEM((E_PER_W,), jnp.int32),
            pltpu.VMEM((B, D), jnp.float32),
            pltpu.VMEM((B, D), jnp.float32),
            pltpu.VMEM((B, D), jnp.float32),
            pltpu.VMEM((B, D), jnp.float32),
            pltpu.VMEM((B, D), jnp.float32),
            pltpu.VMEM((B, D), jnp.float32),
            pltpu.VMEM((B * 17,), jnp.float32),
            pltpu.VMEM((E_PER_W,), jnp.float32),
            pltpu.SemaphoreType.DMA,
            pltpu.SemaphoreType.DMA,
            pltpu.SemaphoreType.DMA,
            pltpu.SemaphoreType.DMA,
            pltpu.SemaphoreType.DMA,
            pltpu.SemaphoreType.DMA,
        ],
    )
    return f(b_tab, node_emb, ai, oi)


def kernel(node_emb, edge_index, edge_type, relations, tokeys, toqueries):
    si2 = edge_index[0].reshape(E_ROWS, D)
    p2 = edge_type.reshape(E_ROWS, D)
    k_tab, ai2 = _project(node_emb, tokeys, si2, p2)
    b_tab = _expand(k_tab, relations, toqueries)
    ai = ai2.reshape(E)
    oi = edge_index[1]
    dots, new_node_emb = _sc_call(b_tab, node_emb, ai, oi)
    return dots, new_node_emb
